# baseline (speedup 1.0000x reference)
import jax
import jax.numpy as jnp
from jax import lax
from jax.experimental import pallas as pl
from jax.experimental.pallas import tpu as pltpu
from jax.experimental.pallas import tpu_sc as plsc

B = 4096
H = 200
D = 64
NW = 32
BB = B // 128
NBUF = 3
EPS = 1e-8
SQRT_D = 8.0
MAGIC = 0x5F3759DF


def _sc_body(idxT_hbm, w_hbm, tab_hbm, out_hbm, idx_all, rows, outT, w_v,
             sem_g, sem_o):
    wid = lax.axis_index("s") * 2 + lax.axis_index("c")
    pltpu.sync_copy(w_hbm, w_v)
    pltpu.sync_copy(idxT_hbm.at[:, pl.ds(wid * 128, 128)], idx_all)

    def start_gather(h, p):
        pltpu.async_copy(tab_hbm.at[idx_all.at[h]], rows.at[p], sem_g.at[p])

    def wait_gather(p):
        pltpu.make_async_copy(tab_hbm.at[idx_all.at[0]], rows.at[p],
                              sem_g.at[p]).wait()

    def compute_panel(p):
        iota = lax.iota(jnp.int32, 16)
        w_regs = [w_v[pl.ds(k * 16, 16)] for k in range(4)]
        jb16 = [(iota + k * 16) >> 3 for k in range(4)]
        js16 = [(iota + k * 16) & 7 for k in range(4)]
        psplat = jnp.full((16,), p, jnp.int32)

        @plsc.parallel_loop(0, 128, 1, unroll=8)
        def row_body(r):
            v = [rows[p, r, pl.ds(k * 16, 16)] for k in range(4)]
            acc = v[0] * v[0] + v[1] * v[1] + v[2] * v[2] + v[3] * v[3]
            for sh in (8, 4, 2, 1):
                perm = jnp.bitwise_xor(iota, sh)
                acc = acc + acc.at[perm].get(mode="promise_in_bounds")
            x = acc + EPS
            bits = lax.bitcast_convert_type(x, jnp.int32)
            y = lax.bitcast_convert_type(
                jnp.full((16,), MAGIC, jnp.int32) - (bits >> 1),
                jnp.float32)
            y = y * (1.5 - 0.5 * x * y * y)
            s = y * SQRT_D
            rsplat = jnp.full((16,), r, jnp.int32)
            for k in range(4):
                plsc.store_scatter(
                    outT, [psplat, jb16[k], js16[k], rsplat],
                    v[k] * (w_regs[k] * s))

    def write_out(h, p):
        for jb in range(8):
            pltpu.async_copy(outT.at[p, jb], out_hbm.at[h, jb, wid],
                             sem_o.at[p])

    def wait_out(p):
        for jb in range(8):
            pltpu.make_async_copy(outT.at[p, jb], out_hbm.at[0, jb, 0],
                                  sem_o.at[p]).wait()

    start_gather(0, 0)
    start_gather(1, 1)

    def panel_body(h, carry):
        p = lax.rem(h, NBUF)

        @pl.when(h + 2 < H)
        def _():
            q = lax.rem(h + 2, NBUF)

            @pl.when(h >= 1)
            def _():
                wait_out(q)
            start_gather(h + 2, q)

        wait_gather(p)
        compute_panel(p)
        write_out(h, p)
        return carry

    lax.fori_loop(0, H, panel_body, 0)
    for p in range(NBUF):
        wait_out(p)


def kernel(input_seq, item_table, rms_weight):
    idxT = input_seq.astype(jnp.int32).T
    tab = jnp.pad(item_table, ((0, 0), (0, D)))
    mesh = plsc.VectorSubcoreMesh(core_axis_name="c", subcore_axis_name="s")
    out5 = pl.kernel(
        _sc_body,
        out_type=jax.ShapeDtypeStruct((H, 8, BB, 8, 128), jnp.float32),
        mesh=mesh,
        compiler_params=pltpu.CompilerParams(needs_layout_passes=False),
        scratch_types=[
            pltpu.VMEM((H, 128), jnp.int32),
            pltpu.VMEM((NBUF, 128, 2 * D), jnp.float32),
            pltpu.VMEM((NBUF, 8, 8, 128), jnp.float32),
            pltpu.VMEM((D,), jnp.float32),
            pltpu.SemaphoreType.DMA((NBUF,)),
            pltpu.SemaphoreType.DMA((NBUF,)),
        ],
    )(idxT, rms_weight, tab)
    return jnp.transpose(out5, (2, 4, 0, 1, 3)).reshape(B, H, D)


# speedup vs baseline: 1.0456x; 1.0456x over previous
import jax
import jax.numpy as jnp
from jax import lax
from jax.experimental import pallas as pl
from jax.experimental.pallas import tpu as pltpu
from jax.experimental.pallas import tpu_sc as plsc

B = 4096
H = 200
D = 64
NW = 32
BB = B // 128
NBUF = 4
EPS = 1e-8
SQRT_D = 8.0
MAGIC = 0x5F3759DF


def _sc_body(idxT_hbm, w_hbm, tab_hbm, out_hbm, idx_all, rows, outT, w_v,
             sem_g, sem_o):
    wid = lax.axis_index("s") * 2 + lax.axis_index("c")
    pltpu.sync_copy(w_hbm, w_v)
    pltpu.sync_copy(idxT_hbm.at[:, pl.ds(wid * 128, 128)], idx_all)

    def start_gather(h, p):
        pltpu.async_copy(tab_hbm.at[idx_all.at[h]], rows.at[p], sem_g.at[p])

    def wait_gather(p):
        pltpu.make_async_copy(tab_hbm.at[idx_all.at[0]], rows.at[p],
                              sem_g.at[p]).wait()

    iota = lax.iota(jnp.int32, 16)
    w8 = [None] * 4

    def compute_panel(p):
        j16 = [iota + k * 16 for k in range(4)]

        @plsc.parallel_loop(0, 128, 1, unroll=8)
        def row_body(r):
            v = [rows[p, r, pl.ds(k * 16, 16)] for k in range(4)]
            acc = v[0] * v[0] + v[1] * v[1] + v[2] * v[2] + v[3] * v[3]
            for sh in (8, 4, 2, 1):
                perm = jnp.bitwise_xor(iota, sh)
                acc = acc + acc.at[perm].get(mode="promise_in_bounds")
            x = acc + EPS
            bits = lax.bitcast_convert_type(x, jnp.int32)
            y = lax.bitcast_convert_type(
                jnp.full((16,), MAGIC, jnp.int32) - (bits >> 1),
                jnp.float32)
            y = y * (1.5 - 0.5 * x * y * y)
            rsplat = jnp.full((16,), r, jnp.int32)
            for k in range(4):
                plsc.store_scatter(outT.at[p], [j16[k], rsplat],
                                   v[k] * (w8[k] * y))

    def write_out(h, p):
        for jb in range(8):
            pltpu.async_copy(outT.at[p, pl.ds(jb * 8, 8)],
                             out_hbm.at[h, jb, wid], sem_o.at[p])

    def wait_out(p):
        for jb in range(8):
            pltpu.make_async_copy(outT.at[p, pl.ds(jb * 8, 8)],
                                  out_hbm.at[0, jb, 0], sem_o.at[p]).wait()

    start_gather(0, 0)
    start_gather(1, 1)

    def group_body(i, carry):
        for p in range(NBUF):
            h = i * NBUF + p

            @pl.when(h + 2 < H)
            def _():
                q = (p + 2) % NBUF

                @pl.when(h >= NBUF - 2)
                def _():
                    wait_out(q)
                start_gather(h + 2, q)

            wait_gather(p)
            compute_panel(p)
            write_out(h, p)
        return carry

    for k in range(4):
        w8[k] = w_v[pl.ds(k * 16, 16)] * SQRT_D
    lax.fori_loop(0, H // NBUF, group_body, 0)
    for p in range(NBUF):
        wait_out(p)


def kernel(input_seq, item_table, rms_weight):
    idxT = input_seq.astype(jnp.int32).T
    tab = jnp.pad(item_table, ((0, 0), (0, D)))
    mesh = plsc.VectorSubcoreMesh(core_axis_name="c", subcore_axis_name="s")
    out5 = pl.kernel(
        _sc_body,
        out_type=jax.ShapeDtypeStruct((H, 8, BB, 8, 128), jnp.float32),
        mesh=mesh,
        compiler_params=pltpu.CompilerParams(needs_layout_passes=False),
        scratch_types=[
            pltpu.VMEM((H, 128), jnp.int32),
            pltpu.VMEM((NBUF, 128, 2 * D), jnp.float32),
            pltpu.VMEM((NBUF, 64, 128), jnp.float32),
            pltpu.VMEM((D,), jnp.float32),
            pltpu.SemaphoreType.DMA((NBUF,)),
            pltpu.SemaphoreType.DMA((NBUF,)),
        ],
    )(idxT, rms_weight, tab)
    return jnp.transpose(out5, (2, 4, 0, 1, 3)).reshape(B, H, D)


# speedup vs baseline: 1.6625x; 1.5900x over previous
import jax
import jax.numpy as jnp
from jax import lax
from jax.experimental import pallas as pl
from jax.experimental.pallas import tpu as pltpu
from jax.experimental.pallas import tpu_sc as plsc

B = 4096
H = 200
D = 64
NW = 32
BB = B // 128
NBUF = 4
SD = 65
EPS = 1e-8
SQRT_D = 8.0
MAGIC = 0x5F3759DF


def _sc_body(idxT_hbm, w_hbm, tab_hbm, out_hbm, idx_all, rows, outD, outT,
             w_v, sem_g, sem_o):
    wid = lax.axis_index("s") * 2 + lax.axis_index("c")
    pltpu.sync_copy(w_hbm, w_v)
    pltpu.sync_copy(idxT_hbm.at[:, pl.ds(wid * 128, 128)], idx_all)

    def start_gather(h, p):
        pltpu.async_copy(tab_hbm.at[idx_all.at[h]], rows.at[p], sem_g.at[p])

    def wait_gather(p):
        pltpu.make_async_copy(tab_hbm.at[idx_all.at[0]], rows.at[p],
                              sem_g.at[p]).wait()

    iota = lax.iota(jnp.int32, 16)
    w8 = [None] * 4

    def phase1(p):
        @plsc.parallel_loop(0, 128, 1, unroll=8)
        def row_body(r):
            v = [rows[p, r, pl.ds(k * 16, 16)] for k in range(4)]
            acc = v[0] * v[0] + v[1] * v[1] + v[2] * v[2] + v[3] * v[3]
            for sh in (8, 4, 2, 1):
                perm = jnp.bitwise_xor(iota, sh)
                acc = acc + acc.at[perm].get(mode="promise_in_bounds")
            x = acc + EPS
            bits = lax.bitcast_convert_type(x, jnp.int32)
            y = lax.bitcast_convert_type(
                jnp.full((16,), MAGIC, jnp.int32) - (bits >> 1),
                jnp.float32)
            y = y * (1.5 - 0.5 * x * y * y)
            base = iota + r * SD
            for k in range(4):
                plsc.store_scatter(outD, [base + k * 16],
                                   v[k] * (w8[k] * y))

    def phase2(ot):
        r65 = [(iota + g * 16) * SD for g in range(8)]

        @plsc.parallel_loop(0, D, 1, unroll=4)
        def col_body(j):
            jsplat = jnp.full((16,), j, jnp.int32)
            for g in range(8):
                c = plsc.load_gather(outD, [r65[g] + jsplat])
                outT[ot, j, pl.ds(g * 16, 16)] = c

    def write_out(h, ot):
        for jb in range(8):
            pltpu.async_copy(outT.at[ot, pl.ds(jb * 8, 8)],
                             out_hbm.at[h, jb, wid], sem_o.at[ot])

    def wait_out(ot):
        for jb in range(8):
            pltpu.make_async_copy(outT.at[ot, pl.ds(jb * 8, 8)],
                                  out_hbm.at[0, jb, 0], sem_o.at[ot]).wait()

    start_gather(0, 0)
    start_gather(1, 1)

    def group_body(i, carry):
        for p in range(NBUF):
            h = i * NBUF + p
            ot = p % 2

            @pl.when(h + 2 < H)
            def _():
                start_gather(h + 2, (p + 2) % NBUF)

            wait_gather(p)
            phase1(p)

            @pl.when(h >= 2)
            def _():
                wait_out(ot)
            phase2(ot)
            write_out(h, ot)
        return carry

    for k in range(4):
        w8[k] = w_v[pl.ds(k * 16, 16)] * SQRT_D
    lax.fori_loop(0, H // NBUF, group_body, 0)
    wait_out(0)
    wait_out(1)


def kernel(input_seq, item_table, rms_weight):
    idxT = input_seq.astype(jnp.int32).T
    tab = jnp.pad(item_table, ((0, 0), (0, D)))
    mesh = plsc.VectorSubcoreMesh(core_axis_name="c", subcore_axis_name="s")
    out5 = pl.kernel(
        _sc_body,
        out_type=jax.ShapeDtypeStruct((H, 8, BB, 8, 128), jnp.float32),
        mesh=mesh,
        compiler_params=pltpu.CompilerParams(needs_layout_passes=False),
        scratch_types=[
            pltpu.VMEM((H, 128), jnp.int32),
            pltpu.VMEM((NBUF, 128, 2 * D), jnp.float32),
            pltpu.VMEM((128 * SD,), jnp.float32),
            pltpu.VMEM((2, 64, 128), jnp.float32),
            pltpu.VMEM((D,), jnp.float32),
            pltpu.SemaphoreType.DMA((NBUF,)),
            pltpu.SemaphoreType.DMA((2,)),
        ],
    )(idxT, rms_weight, tab)
    return jnp.transpose(out5, (2, 4, 0, 1, 3)).reshape(B, H, D)


# speedup vs baseline: 1.6900x; 1.0165x over previous
import jax
import jax.numpy as jnp
from jax import lax
from jax.experimental import pallas as pl
from jax.experimental.pallas import tpu as pltpu
from jax.experimental.pallas import tpu_sc as plsc

B = 4096
H = 200
D = 64
NW = 32
BB = B // 128
NBUF = 4
SD = 65
EPS = 1e-8
SQRT_D = 8.0
MAGIC = 0x5F3759DF


def _sc_body(idxT_hbm, w_hbm, tab_hbm, out_hbm, idx_all, rows, outD, outT,
             w_v, sem_g, sem_o):
    wid = lax.axis_index("s") * 2 + lax.axis_index("c")
    pltpu.sync_copy(w_hbm, w_v)
    pltpu.sync_copy(idxT_hbm.at[:, pl.ds(wid * 128, 128)], idx_all)

    def start_gather(h, p):
        pltpu.async_copy(tab_hbm.at[idx_all.at[h]], rows.at[p], sem_g.at[p])

    def wait_gather(p):
        pltpu.make_async_copy(tab_hbm.at[idx_all.at[0]], rows.at[p],
                              sem_g.at[p]).wait()

    iota = lax.iota(jnp.int32, 16)
    w8 = [None] * 4

    def phase1(p):
        @plsc.parallel_loop(0, 128, 1, unroll=8)
        def row_body(r):
            v = [rows[p, r, pl.ds(k * 16, 16)] for k in range(4)]
            acc = v[0] * v[0] + v[1] * v[1] + v[2] * v[2] + v[3] * v[3]
            for sh in (8, 4, 2, 1):
                perm = jnp.bitwise_xor(iota, sh)
                acc = acc + acc.at[perm].get(mode="promise_in_bounds")
            x = acc + EPS
            bits = lax.bitcast_convert_type(x, jnp.int32)
            y = lax.bitcast_convert_type(
                jnp.full((16,), MAGIC, jnp.int32) - (bits >> 1),
                jnp.float32)
            y = y * (1.5 - 0.5 * x * y * y)
            base = iota + r * SD
            for k in range(4):
                plsc.store_scatter(outD, [base + k * 16],
                                   v[k] * (w8[k] * y))

    def phase2(ot):
        r65 = [(iota + g * 16) * SD for g in range(8)]

        @plsc.parallel_loop(0, D, 1, unroll=8)
        def col_body(j):
            jsplat = jnp.full((16,), j, jnp.int32)
            for g in range(8):
                c = plsc.load_gather(outD, [r65[g] + jsplat])
                outT[ot, j, pl.ds(g * 16, 16)] = c

    def write_out(h, ot):
        for jb in range(8):
            pltpu.async_copy(outT.at[ot, pl.ds(jb * 8, 8)],
                             out_hbm.at[h, jb, wid], sem_o.at[ot])

    def wait_out(ot):
        for jb in range(8):
            pltpu.make_async_copy(outT.at[ot, pl.ds(jb * 8, 8)],
                                  out_hbm.at[0, jb, 0], sem_o.at[ot]).wait()

    start_gather(0, 0)
    start_gather(1, 1)
    start_gather(2, 2)

    def group_body(i, carry):
        for p in range(NBUF):
            h = i * NBUF + p
            ot = p % 2

            @pl.when(h + 3 < H)
            def _():
                start_gather(h + 3, (p + 3) % NBUF)

            wait_gather(p)
            phase1(p)

            @pl.when(h >= 2)
            def _():
                wait_out(ot)
            phase2(ot)
            write_out(h, ot)
        return carry

    for k in range(4):
        w8[k] = w_v[pl.ds(k * 16, 16)] * SQRT_D
    lax.fori_loop(0, H // NBUF, group_body, 0)
    wait_out(0)
    wait_out(1)


def kernel(input_seq, item_table, rms_weight):
    idxT = input_seq.astype(jnp.int32).T
    tab = jnp.pad(item_table, ((0, 0), (0, D)))
    mesh = plsc.VectorSubcoreMesh(core_axis_name="c", subcore_axis_name="s")
    out5 = pl.kernel(
        _sc_body,
        out_type=jax.ShapeDtypeStruct((H, 8, BB, 8, 128), jnp.float32),
        mesh=mesh,
        compiler_params=pltpu.CompilerParams(needs_layout_passes=False),
        scratch_types=[
            pltpu.VMEM((H, 128), jnp.int32),
            pltpu.VMEM((NBUF, 128, 2 * D), jnp.float32),
            pltpu.VMEM((128 * SD,), jnp.float32),
            pltpu.VMEM((2, 64, 128), jnp.float32),
            pltpu.VMEM((D,), jnp.float32),
            pltpu.SemaphoreType.DMA((NBUF,)),
            pltpu.SemaphoreType.DMA((2,)),
        ],
    )(idxT, rms_weight, tab)
    return jnp.transpose(out5, (2, 4, 0, 1, 3)).reshape(B, H, D)
